# baseline (speedup 1.0000x reference)
import functools

import jax
import jax.numpy as jnp
from jax import lax
from jax.experimental import pallas as pl
from jax.experimental.pallas import tpu as pltpu
from jax.experimental.pallas import tpu_sc as plsc

N = 10000
EG = 320000
IN_DIM = 128
D = 16
H = 8
HD = H * D
CLAMP = 5.0

NC = 2
NS = 16
NW = NC * NS

CH = 128
NCHUNK = EG // CH
DUMP_WORKERS = 10
ROWS_PER_SUB = N // DUMP_WORKERS
INIT_ROWS = 200

_F32 = jnp.float32
_HIGH = lax.Precision.HIGHEST


def _proj_body(x_ref, qw_ref, kw_ref, vw_ref, qb_ref, kb_ref, vb_ref,
               q_ref, k_ref, v_ref):
    x = x_ref[...]
    q_ref[...] = jnp.dot(x, qw_ref[...], precision=_HIGH) + qb_ref[...]
    k_ref[...] = jnp.dot(x, kw_ref[...], precision=_HIGH) + kb_ref[...]
    v_ref[...] = jnp.dot(x, vw_ref[...], precision=_HIGH) + vb_ref[...]


def _project(x, QwT, KwT, VwT, Qb2, Kb2, Vb2):
    blk = 1000
    grid = N // blk
    wspec = pl.BlockSpec((IN_DIM, HD), lambda i: (0, 0))
    bspec = pl.BlockSpec((1, HD), lambda i: (0, 0))
    nspec = pl.BlockSpec((blk, HD), lambda i: (i, 0))
    return pl.pallas_call(
        _proj_body,
        grid=(grid,),
        in_specs=[pl.BlockSpec((blk, IN_DIM), lambda i: (i, 0)),
                  wspec, wspec, wspec, bspec, bspec, bspec],
        out_specs=[nspec, nspec, nspec],
        out_shape=[jax.ShapeDtypeStruct((N, HD), _F32)] * 3,
        compiler_params=pltpu.CompilerParams(
            dimension_semantics=("parallel",)),
    )(x, QwT, KwT, VwT, Qb2, Kb2, Vb2)


def _gather_edges(q_h, k_h, v_h, src, dst):
    ne = src.shape[0]
    ncnk = ne // CH
    mesh = plsc.VectorSubcoreMesh(core_axis_name="c", subcore_axis_name="s")
    nj = (ncnk + NW - 1) // NW

    npairs = (nj + 1) // 2

    @functools.partial(
        pl.kernel,
        mesh=mesh,
        out_type=[jax.ShapeDtypeStruct((ne, HD), _F32)] * 2,
        scratch_types=[
            pltpu.VMEM((CH,), jnp.int32),
            pltpu.VMEM((CH,), jnp.int32),
            pltpu.VMEM((CH,), jnp.int32),
            pltpu.VMEM((CH,), jnp.int32),
            pltpu.VMEM((CH, HD), _F32),
            pltpu.VMEM((CH, HD), _F32),
            pltpu.VMEM((CH, HD), _F32),
            pltpu.VMEM((CH, HD), _F32),
            pltpu.VMEM((CH, HD), _F32),
            pltpu.VMEM((CH, HD), _F32),
            pltpu.SemaphoreType.DMA,
            pltpu.SemaphoreType.DMA,
            pltpu.SemaphoreType.DMA,
        ],
    )
    def _kern(qh, kh, vh, src_h, dst_h, kq_out, v_out,
              si_a, di_a, si_b, di_b, bk_a, bq_a, bv_a, bk_b, bq_b, bv_b,
              sem_a, sem_b, sem_s):
        w = lax.axis_index("s") * NC + lax.axis_index("c")

        def _issue(t, si, di, bk, bq, bv, sem):
            base = t * CH
            pltpu.sync_copy(src_h.at[pl.ds(base, CH)], si)
            pltpu.sync_copy(dst_h.at[pl.ds(base, CH)], di)
            pltpu.async_copy(kh.at[si], bk, sem)
            pltpu.async_copy(qh.at[di], bq, sem)
            pltpu.async_copy(vh.at[si], bv, sem)

        def _consume(t, si, di, bk, bq, bv, sem):
            base = t * CH
            pltpu.make_async_copy(kh.at[si], bk, sem).wait()
            pltpu.make_async_copy(qh.at[di], bq, sem).wait()
            pltpu.make_async_copy(vh.at[si], bv, sem).wait()

            @pl.loop(0, CH)
            def _(r):
                for g in range(HD // 16):
                    bk[r, pl.ds(g * 16, 16)] = (
                        bk[r, pl.ds(g * 16, 16)] + bq[r, pl.ds(g * 16, 16)])

            kq_slice = kq_out.at[pl.ds(base, CH)]
            v_slice = v_out.at[pl.ds(base, CH)]
            pltpu.async_copy(bk, kq_slice, sem_s)
            pltpu.async_copy(bv, v_slice, sem_s)
            pltpu.make_async_copy(bk, kq_slice, sem_s).wait()
            pltpu.make_async_copy(bv, v_slice, sem_s).wait()

        _issue(w, si_a, di_a, bk_a, bq_a, bv_a, sem_a)

        @pl.loop(0, npairs)
        def _(q):
            ta = (2 * q) * NW + w
            tb = ta + NW
            ta2 = tb + NW

            @pl.when(tb < ncnk)
            def _():
                _issue(tb, si_b, di_b, bk_b, bq_b, bv_b, sem_b)

            @pl.when(ta < ncnk)
            def _():
                _consume(ta, si_a, di_a, bk_a, bq_a, bv_a, sem_a)

            @pl.when(ta2 < ncnk)
            def _():
                _issue(ta2, si_a, di_a, bk_a, bq_a, bv_a, sem_a)

            @pl.when(tb < ncnk)
            def _():
                _consume(tb, si_b, di_b, bk_b, bq_b, bv_b, sem_b)

    return _kern(q_h, k_h, v_h, src, dst)


def _signed_sqrt(v):
    return jnp.sign(v) * jnp.sqrt(jnp.abs(v))


def _edge_body(ea_ref, kq_ref, v_ref, eww_ref, ewb_ref, ebw_ref, ebb_ref,
               awm_ref, rep_ref, et_prev_ref, et_ref, msg_ref, wep_ref,
               p_ref):
    del et_prev_ref
    ea = ea_ref[...]
    e_w = jnp.dot(ea, eww_ref[...]) + ebw_ref[...]
    e_b2 = jnp.dot(ea, ewb_ref[...]) + ebb_ref[...]
    et = _signed_sqrt(kq_ref[...] * e_w) + e_b2
    et_ref[...] = et
    s16 = jnp.dot(et, awm_ref[...], precision=_HIGH)
    p16 = jnp.exp(jnp.clip(s16, -CLAMP, CLAMP))
    pbig = jnp.dot(p16, rep_ref[...], precision=_HIGH)
    p_ref[...] = pbig
    msg_ref[...] = v_ref[...] * pbig
    wep_ref[...] = et * pbig


def _edge_stage(edge_attr, kq, vsrc, EwwT, EwbT, Ebw2, Ebb2, AwM, Rep,
                et_prev, blk0):
    blk = 2000
    ne = kq.shape[0]
    grid = ne // blk
    espec = pl.BlockSpec((blk, HD), lambda i: (i, 0))
    easpec = pl.BlockSpec((blk, HD), lambda i: (i + blk0, 0))
    return pl.pallas_call(
        _edge_body,
        grid=(grid,),
        in_specs=[easpec, espec, espec,
                  pl.BlockSpec((IN_DIM, HD), lambda i: (0, 0)),
                  pl.BlockSpec((IN_DIM, HD), lambda i: (0, 0)),
                  pl.BlockSpec((1, HD), lambda i: (0, 0)),
                  pl.BlockSpec((1, HD), lambda i: (0, 0)),
                  pl.BlockSpec((HD, 16), lambda i: (0, 0)),
                  pl.BlockSpec((16, HD), lambda i: (0, 0)),
                  easpec],
        out_specs=[easpec, espec, espec, espec],
        out_shape=[jax.ShapeDtypeStruct((EG, HD), _F32)] +
                  [jax.ShapeDtypeStruct((ne, HD), _F32)] * 3,
        input_output_aliases={9: 0},
        compiler_params=pltpu.CompilerParams(
            dimension_semantics=("parallel",)),
    )(edge_attr, kq, vsrc, EwwT, EwbT, Ebw2, Ebb2, AwM, Rep, et_prev)


NH = N // 2
ACC_ROWS = NH + 8


def _seg_sum_128(stream, dst):
    ne = stream.shape[0]
    ncnk = ne // CH
    mesh = plsc.VectorSubcoreMesh(core_axis_name="c", subcore_axis_name="s")
    nj = (ncnk + NS - 1) // NS
    ndw = NH // ROWS_PER_SUB

    npairs = (nj + 1) // 2

    @functools.partial(
        pl.kernel,
        mesh=mesh,
        out_type=jax.ShapeDtypeStruct((N, HD), _F32),
        scratch_types=[
            pltpu.VMEM((CH,), jnp.int32),
            pltpu.VMEM((CH,), jnp.int32),
            pltpu.VMEM((CH,), jnp.int32),
            pltpu.VMEM((CH, HD), _F32),
            pltpu.VMEM((CH, HD), _F32),
            pltpu.VMEM((INIT_ROWS, HD), _F32),
            pltpu.VMEM_SHARED((ACC_ROWS, HD), _F32),
            pltpu.SemaphoreType.DMA,
            pltpu.SemaphoreType.DMA,
        ],
    )
    def _kern(stream_h, dst_h, acc_o, didx_a, didx_b, didx2, dbuf_a, dbuf_b,
              zbuf, acc, sem_a, sem_b):
        c = lax.axis_index("c")
        s = lax.axis_index("s")
        lo = c * NH
        zero = jnp.zeros((16,), _F32)

        @pl.loop(0, INIT_ROWS)
        def _(r):
            for g in range(HD // 16):
                zbuf[r, pl.ds(g * 16, 16)] = zero

        @pl.when(s < ndw)
        def _():
            @pl.loop(0, ROWS_PER_SUB // INIT_ROWS)
            def _(q):
                row0 = s * ROWS_PER_SUB + q * INIT_ROWS
                pltpu.sync_copy(zbuf, acc.at[pl.ds(row0, INIT_ROWS)])

        plsc.subcore_barrier()

        def _issue(t, didx, dbuf, sem):
            base = t * CH
            pltpu.async_copy(dst_h.at[pl.ds(base, CH)], didx, sem)
            pltpu.async_copy(stream_h.at[pl.ds(base, CH)], dbuf, sem)

        def _consume(t, didx, dbuf, sem):
            base = t * CH
            pltpu.make_async_copy(dst_h.at[pl.ds(base, CH)], didx, sem).wait()
            pltpu.make_async_copy(stream_h.at[pl.ds(base, CH)], dbuf,
                                  sem).wait()

            @pl.loop(0, CH, step=16)
            def _(g):
                sl = pl.ds(g, 16)
                loc = didx[sl] - lo
                ok = (loc >= 0) & (loc < NH)
                didx2[sl] = jnp.where(ok, loc, NH)

            pltpu.sync_copy(dbuf, acc.at[didx2], add=True)

        _issue(s, didx_a, dbuf_a, sem_a)

        @pl.loop(0, npairs)
        def _(q):
            ta = (2 * q) * NS + s
            tb = ta + NS
            ta2 = tb + NS

            @pl.when(tb < ncnk)
            def _():
                _issue(tb, didx_b, dbuf_b, sem_b)

            @pl.when(ta < ncnk)
            def _():
                _consume(ta, didx_a, dbuf_a, sem_a)

            @pl.when(ta2 < ncnk)
            def _():
                _issue(ta2, didx_a, dbuf_a, sem_a)

            @pl.when(tb < ncnk)
            def _():
                _consume(tb, didx_b, dbuf_b, sem_b)

        plsc.subcore_barrier()

        @pl.when(s < ndw)
        def _():
            row0 = s * ROWS_PER_SUB
            pltpu.sync_copy(acc.at[pl.ds(row0, ROWS_PER_SUB)],
                            acc_o.at[pl.ds(lo + row0, ROWS_PER_SUB)])

    return _kern(stream, dst)


def _segment_sums(msg, wep, pbig, dst):
    return (_seg_sum_128(msg, dst), _seg_sum_128(pbig, dst),
            _seg_sum_128(wep, dst))


def _final_body(av1_ref, as1_ref, ae1_ref, av2_ref, as2_ref, ae2_ref,
                vem_ref, out_ref):
    ssum = as1_ref[...] + as2_ref[...] + 1e-16
    rv = (ae1_ref[...] + ae2_ref[...]) / ssum
    out_ref[...] = ((av1_ref[...] + av2_ref[...]) / ssum +
                    jnp.dot(rv, vem_ref[...], precision=_HIGH))


def _finalize(accs6, VeM):
    blk = 1000
    grid = N // blk
    nspec = pl.BlockSpec((blk, HD), lambda i: (i, 0))
    return pl.pallas_call(
        _final_body,
        grid=(grid,),
        in_specs=[nspec] * 6 + [pl.BlockSpec((HD, HD), lambda i: (0, 0))],
        out_specs=nspec,
        out_shape=jax.ShapeDtypeStruct((N, HD), _F32),
        compiler_params=pltpu.CompilerParams(
            dimension_semantics=("parallel",)),
    )(*accs6, VeM)


def kernel(x, edge_attr, edge_index, Qw, Qb, Kw, Kb, Ew, Eb, Vw, Vb, Aw,
           VeRow):
    hs = jnp.arange(H)
    ds_ = jnp.arange(D)
    perm_w = (hs[:, None] * 2 * D + ds_[None, :]).reshape(-1)
    perm_b = (hs[:, None] * 2 * D + D + ds_[None, :]).reshape(-1)

    QwT = Qw.T
    KwT = Kw.T
    VwT = Vw.T
    Qb2 = Qb[None, :]
    Kb2 = Kb[None, :]
    Vb2 = Vb[None, :]
    EwwT = Ew[perm_w, :].T
    EwbT = Ew[perm_b, :].T
    Ebw2 = Eb[perm_w][None, :]
    Ebb2 = Eb[perm_b][None, :]

    rows = (hs[:, None] * D + ds_[None, :]).reshape(-1)
    AwM = jnp.zeros((HD, 16), _F32).at[
        rows, jnp.repeat(hs, D)].set(Aw[:, :, 0].T.reshape(-1))
    Rep = jnp.zeros((16, HD), _F32).at[jnp.repeat(hs, D), rows].set(1.0)
    cs = (hs[:, None, None] * D + jnp.zeros((D, D), jnp.int32)[None] +
          jnp.arange(D)[None, None, :])
    rs = (hs[:, None, None] * D + ds_[None, :, None] +
          jnp.zeros((D, D), jnp.int32)[None] * 0)
    VeM = jnp.zeros((HD, HD), _F32).at[
        rs.reshape(-1), cs.reshape(-1)].set(
            jnp.transpose(VeRow, (1, 0, 2)).reshape(-1))

    src = edge_index[0]
    dst = edge_index[1]

    q_h, k_h, v_h = _project(x, QwT, KwT, VwT, Qb2, Kb2, Vb2)

    eh = EG // 2
    et = jnp.zeros((EG, HD), _F32)
    accs6 = [None] * 6
    for half in (0, 1):
        src_h = lax.slice(src, (half * eh,), ((half + 1) * eh,))
        dst_h = lax.slice(dst, (half * eh,), ((half + 1) * eh,))
        kq, vsrc = _gather_edges(q_h, k_h, v_h, src_h, dst_h)
        et, msg, wep, pbig = _edge_stage(edge_attr, kq, vsrc,
                                         EwwT, EwbT, Ebw2, Ebb2, AwM, Rep,
                                         et, half * (eh // 2000))
        accs6[half * 3 + 0] = _seg_sum_128(msg, dst_h)
        accs6[half * 3 + 1] = _seg_sum_128(pbig, dst_h)
        accs6[half * 3 + 2] = _seg_sum_128(wep, dst_h)

    out = _finalize(accs6, VeM)
    return (out.reshape(N, H, D), et)


# speedup vs baseline: 1.0737x; 1.0737x over previous
import functools

import jax
import jax.numpy as jnp
from jax import lax
from jax.experimental import pallas as pl
from jax.experimental.pallas import tpu as pltpu
from jax.experimental.pallas import tpu_sc as plsc

N = 10000
EG = 320000
IN_DIM = 128
D = 16
H = 8
HD = H * D
CLAMP = 5.0

NC = 2
NS = 16
NW = NC * NS

CH = 128
NCHUNK = EG // CH
DUMP_WORKERS = 10
ROWS_PER_SUB = N // DUMP_WORKERS
INIT_ROWS = 200

_F32 = jnp.float32
_HIGH = lax.Precision.HIGHEST


def _proj_body(x_ref, qw_ref, kw_ref, vw_ref, qb_ref, kb_ref, vb_ref,
               q_ref, k_ref, v_ref):
    x = x_ref[...]
    q_ref[...] = jnp.dot(x, qw_ref[...], precision=_HIGH) + qb_ref[...]
    k_ref[...] = jnp.dot(x, kw_ref[...], precision=_HIGH) + kb_ref[...]
    v_ref[...] = jnp.dot(x, vw_ref[...], precision=_HIGH) + vb_ref[...]


def _project(x, QwT, KwT, VwT, Qb2, Kb2, Vb2):
    blk = 1000
    grid = N // blk
    wspec = pl.BlockSpec((IN_DIM, HD), lambda i: (0, 0))
    bspec = pl.BlockSpec((1, HD), lambda i: (0, 0))
    nspec = pl.BlockSpec((blk, HD), lambda i: (i, 0))
    return pl.pallas_call(
        _proj_body,
        grid=(grid,),
        in_specs=[pl.BlockSpec((blk, IN_DIM), lambda i: (i, 0)),
                  wspec, wspec, wspec, bspec, bspec, bspec],
        out_specs=[nspec, nspec, nspec],
        out_shape=[jax.ShapeDtypeStruct((N, HD), _F32)] * 3,
        compiler_params=pltpu.CompilerParams(
            dimension_semantics=("parallel",)),
    )(x, QwT, KwT, VwT, Qb2, Kb2, Vb2)


def _gather_edges(q_h, k_h, v_h, src, dst):
    ne = src.shape[0]
    ncnk = ne // CH
    mesh = plsc.VectorSubcoreMesh(core_axis_name="c", subcore_axis_name="s")
    nj = (ncnk + NW - 1) // NW

    npairs = (nj + 1) // 2

    @functools.partial(
        pl.kernel,
        mesh=mesh,
        out_type=[jax.ShapeDtypeStruct((ne, HD), _F32)] * 2,
        scratch_types=[
            pltpu.VMEM((CH,), jnp.int32),
            pltpu.VMEM((CH,), jnp.int32),
            pltpu.VMEM((CH,), jnp.int32),
            pltpu.VMEM((CH,), jnp.int32),
            pltpu.VMEM((CH, HD), _F32),
            pltpu.VMEM((CH, HD), _F32),
            pltpu.VMEM((CH, HD), _F32),
            pltpu.VMEM((CH, HD), _F32),
            pltpu.VMEM((CH, HD), _F32),
            pltpu.VMEM((CH, HD), _F32),
            pltpu.SemaphoreType.DMA,
            pltpu.SemaphoreType.DMA,
            pltpu.SemaphoreType.DMA,
        ],
    )
    def _kern(qh, kh, vh, src_h, dst_h, kq_out, v_out,
              si_a, di_a, si_b, di_b, bk_a, bq_a, bv_a, bk_b, bq_b, bv_b,
              sem_a, sem_b, sem_s):
        w = lax.axis_index("s") * NC + lax.axis_index("c")

        def _issue(t, si, di, bk, bq, bv, sem):
            base = t * CH
            pltpu.sync_copy(src_h.at[pl.ds(base, CH)], si)
            pltpu.sync_copy(dst_h.at[pl.ds(base, CH)], di)
            pltpu.async_copy(kh.at[si], bk, sem)
            pltpu.async_copy(qh.at[di], bq, sem)
            pltpu.async_copy(vh.at[si], bv, sem)

        def _consume(t, si, di, bk, bq, bv, sem):
            base = t * CH
            pltpu.make_async_copy(kh.at[si], bk, sem).wait()
            pltpu.make_async_copy(qh.at[di], bq, sem).wait()
            pltpu.make_async_copy(vh.at[si], bv, sem).wait()

            @pl.loop(0, CH)
            def _(r):
                for g in range(HD // 16):
                    bk[r, pl.ds(g * 16, 16)] = (
                        bk[r, pl.ds(g * 16, 16)] + bq[r, pl.ds(g * 16, 16)])

            kq_slice = kq_out.at[pl.ds(base, CH)]
            v_slice = v_out.at[pl.ds(base, CH)]
            pltpu.async_copy(bk, kq_slice, sem_s)
            pltpu.async_copy(bv, v_slice, sem_s)
            pltpu.make_async_copy(bk, kq_slice, sem_s).wait()
            pltpu.make_async_copy(bv, v_slice, sem_s).wait()

        _issue(w, si_a, di_a, bk_a, bq_a, bv_a, sem_a)

        @pl.loop(0, npairs)
        def _(q):
            ta = (2 * q) * NW + w
            tb = ta + NW
            ta2 = tb + NW

            @pl.when(tb < ncnk)
            def _():
                _issue(tb, si_b, di_b, bk_b, bq_b, bv_b, sem_b)

            @pl.when(ta < ncnk)
            def _():
                _consume(ta, si_a, di_a, bk_a, bq_a, bv_a, sem_a)

            @pl.when(ta2 < ncnk)
            def _():
                _issue(ta2, si_a, di_a, bk_a, bq_a, bv_a, sem_a)

            @pl.when(tb < ncnk)
            def _():
                _consume(tb, si_b, di_b, bk_b, bq_b, bv_b, sem_b)

    return _kern(q_h, k_h, v_h, src, dst)


def _signed_sqrt(v):
    return jnp.sign(v) * jnp.sqrt(jnp.abs(v))


def _edge_body(ea_ref, kq_ref, v_ref, eww_ref, ewb_ref, ebw_ref, ebb_ref,
               awm_ref, rep_ref, et_ref, msg_ref, wep_ref, p_ref):
    ea = ea_ref[...]
    e_w = jnp.dot(ea, eww_ref[...]) + ebw_ref[...]
    e_b2 = jnp.dot(ea, ewb_ref[...]) + ebb_ref[...]
    et = _signed_sqrt(kq_ref[...] * e_w) + e_b2
    et_ref[...] = et
    s16 = jnp.dot(et, awm_ref[...], precision=_HIGH)
    p16 = jnp.exp(jnp.clip(s16, -CLAMP, CLAMP))
    pbig = jnp.dot(p16, rep_ref[...], precision=_HIGH)
    p_ref[...] = pbig
    msg_ref[...] = v_ref[...] * pbig
    wep_ref[...] = et * pbig


def _edge_stage(edge_attr, kq, vsrc, EwwT, EwbT, Ebw2, Ebb2, AwM, Rep,
                blk0):
    blk = 2000
    ne = kq.shape[0]
    grid = ne // blk
    espec = pl.BlockSpec((blk, HD), lambda i: (i, 0))
    easpec = pl.BlockSpec((blk, HD), lambda i: (i + blk0, 0))
    return pl.pallas_call(
        _edge_body,
        grid=(grid,),
        in_specs=[easpec, espec, espec,
                  pl.BlockSpec((IN_DIM, HD), lambda i: (0, 0)),
                  pl.BlockSpec((IN_DIM, HD), lambda i: (0, 0)),
                  pl.BlockSpec((1, HD), lambda i: (0, 0)),
                  pl.BlockSpec((1, HD), lambda i: (0, 0)),
                  pl.BlockSpec((HD, 16), lambda i: (0, 0)),
                  pl.BlockSpec((16, HD), lambda i: (0, 0))],
        out_specs=[espec, espec, espec, espec],
        out_shape=[jax.ShapeDtypeStruct((ne, HD), _F32)] * 4,
        compiler_params=pltpu.CompilerParams(
            dimension_semantics=("parallel",)),
    )(edge_attr, kq, vsrc, EwwT, EwbT, Ebw2, Ebb2, AwM, Rep)


NH = N // 2
ACC_ROWS = NH + 8


def _seg_sum_128(stream, dst):
    ne = stream.shape[0]
    ncnk = ne // CH
    mesh = plsc.VectorSubcoreMesh(core_axis_name="c", subcore_axis_name="s")
    nj = (ncnk + NS - 1) // NS
    ndw = NH // ROWS_PER_SUB

    npairs = (nj + 1) // 2

    @functools.partial(
        pl.kernel,
        mesh=mesh,
        out_type=jax.ShapeDtypeStruct((N, HD), _F32),
        scratch_types=[
            pltpu.VMEM((CH,), jnp.int32),
            pltpu.VMEM((CH,), jnp.int32),
            pltpu.VMEM((CH,), jnp.int32),
            pltpu.VMEM((CH, HD), _F32),
            pltpu.VMEM((CH, HD), _F32),
            pltpu.VMEM((INIT_ROWS, HD), _F32),
            pltpu.VMEM_SHARED((ACC_ROWS, HD), _F32),
            pltpu.SemaphoreType.DMA,
            pltpu.SemaphoreType.DMA,
        ],
    )
    def _kern(stream_h, dst_h, acc_o, didx_a, didx_b, didx2, dbuf_a, dbuf_b,
              zbuf, acc, sem_a, sem_b):
        c = lax.axis_index("c")
        s = lax.axis_index("s")
        lo = c * NH
        zero = jnp.zeros((16,), _F32)

        @pl.loop(0, INIT_ROWS)
        def _(r):
            for g in range(HD // 16):
                zbuf[r, pl.ds(g * 16, 16)] = zero

        @pl.when(s < ndw)
        def _():
            @pl.loop(0, ROWS_PER_SUB // INIT_ROWS)
            def _(q):
                row0 = s * ROWS_PER_SUB + q * INIT_ROWS
                pltpu.sync_copy(zbuf, acc.at[pl.ds(row0, INIT_ROWS)])

        plsc.subcore_barrier()

        def _issue(t, didx, dbuf, sem):
            base = t * CH
            pltpu.async_copy(dst_h.at[pl.ds(base, CH)], didx, sem)
            pltpu.async_copy(stream_h.at[pl.ds(base, CH)], dbuf, sem)

        def _consume(t, didx, dbuf, sem):
            base = t * CH
            pltpu.make_async_copy(dst_h.at[pl.ds(base, CH)], didx, sem).wait()
            pltpu.make_async_copy(stream_h.at[pl.ds(base, CH)], dbuf,
                                  sem).wait()

            @pl.loop(0, CH, step=16)
            def _(g):
                sl = pl.ds(g, 16)
                loc = didx[sl] - lo
                ok = (loc >= 0) & (loc < NH)
                didx2[sl] = jnp.where(ok, loc, NH)

            pltpu.sync_copy(dbuf, acc.at[didx2], add=True)

        _issue(s, didx_a, dbuf_a, sem_a)

        @pl.loop(0, npairs)
        def _(q):
            ta = (2 * q) * NS + s
            tb = ta + NS
            ta2 = tb + NS

            @pl.when(tb < ncnk)
            def _():
                _issue(tb, didx_b, dbuf_b, sem_b)

            @pl.when(ta < ncnk)
            def _():
                _consume(ta, didx_a, dbuf_a, sem_a)

            @pl.when(ta2 < ncnk)
            def _():
                _issue(ta2, didx_a, dbuf_a, sem_a)

            @pl.when(tb < ncnk)
            def _():
                _consume(tb, didx_b, dbuf_b, sem_b)

        plsc.subcore_barrier()

        @pl.when(s < ndw)
        def _():
            row0 = s * ROWS_PER_SUB
            pltpu.sync_copy(acc.at[pl.ds(row0, ROWS_PER_SUB)],
                            acc_o.at[pl.ds(lo + row0, ROWS_PER_SUB)])

    return _kern(stream, dst)


def _segment_sums(msg, wep, pbig, dst):
    return (_seg_sum_128(msg, dst), _seg_sum_128(pbig, dst),
            _seg_sum_128(wep, dst))


def _final_body(av1_ref, as1_ref, ae1_ref, av2_ref, as2_ref, ae2_ref,
                vem_ref, out_ref):
    ssum = as1_ref[...] + as2_ref[...] + 1e-16
    rv = (ae1_ref[...] + ae2_ref[...]) / ssum
    out_ref[...] = ((av1_ref[...] + av2_ref[...]) / ssum +
                    jnp.dot(rv, vem_ref[...], precision=_HIGH))


def _finalize(accs6, VeM):
    blk = 1000
    grid = N // blk
    nspec = pl.BlockSpec((blk, HD), lambda i: (i, 0))
    return pl.pallas_call(
        _final_body,
        grid=(grid,),
        in_specs=[nspec] * 6 + [pl.BlockSpec((HD, HD), lambda i: (0, 0))],
        out_specs=nspec,
        out_shape=jax.ShapeDtypeStruct((N, HD), _F32),
        compiler_params=pltpu.CompilerParams(
            dimension_semantics=("parallel",)),
    )(*accs6, VeM)


def kernel(x, edge_attr, edge_index, Qw, Qb, Kw, Kb, Ew, Eb, Vw, Vb, Aw,
           VeRow):
    hs = jnp.arange(H)
    ds_ = jnp.arange(D)
    perm_w = (hs[:, None] * 2 * D + ds_[None, :]).reshape(-1)
    perm_b = (hs[:, None] * 2 * D + D + ds_[None, :]).reshape(-1)

    QwT = Qw.T
    KwT = Kw.T
    VwT = Vw.T
    Qb2 = Qb[None, :]
    Kb2 = Kb[None, :]
    Vb2 = Vb[None, :]
    EwwT = Ew[perm_w, :].T
    EwbT = Ew[perm_b, :].T
    Ebw2 = Eb[perm_w][None, :]
    Ebb2 = Eb[perm_b][None, :]

    rows = (hs[:, None] * D + ds_[None, :]).reshape(-1)
    AwM = jnp.zeros((HD, 16), _F32).at[
        rows, jnp.repeat(hs, D)].set(Aw[:, :, 0].T.reshape(-1))
    Rep = jnp.zeros((16, HD), _F32).at[jnp.repeat(hs, D), rows].set(1.0)
    cs = (hs[:, None, None] * D + jnp.zeros((D, D), jnp.int32)[None] +
          jnp.arange(D)[None, None, :])
    rs = (hs[:, None, None] * D + ds_[None, :, None] +
          jnp.zeros((D, D), jnp.int32)[None] * 0)
    VeM = jnp.zeros((HD, HD), _F32).at[
        rs.reshape(-1), cs.reshape(-1)].set(
            jnp.transpose(VeRow, (1, 0, 2)).reshape(-1))

    src = edge_index[0]
    dst = edge_index[1]

    q_h, k_h, v_h = _project(x, QwT, KwT, VwT, Qb2, Kb2, Vb2)

    eh = EG // 2
    ets = []
    accs6 = [None] * 6
    for half in (0, 1):
        src_h = lax.slice(src, (half * eh,), ((half + 1) * eh,))
        dst_h = lax.slice(dst, (half * eh,), ((half + 1) * eh,))
        kq, vsrc = _gather_edges(q_h, k_h, v_h, src_h, dst_h)
        et, msg, wep, pbig = _edge_stage(edge_attr, kq, vsrc,
                                         EwwT, EwbT, Ebw2, Ebb2, AwM, Rep,
                                         half * (eh // 2000))
        ets.append(et)
        accs6[half * 3 + 0] = _seg_sum_128(msg, dst_h)
        accs6[half * 3 + 1] = _seg_sum_128(pbig, dst_h)
        accs6[half * 3 + 2] = _seg_sum_128(wep, dst_h)

    out = _finalize(accs6, VeM)
    et = jnp.concatenate(ets, axis=0)
    return (out.reshape(N, H, D), et)
